# baseline (speedup 1.0000x reference)
import functools

import jax
import jax.numpy as jnp
from jax.experimental import pallas as pl
from jax.experimental.pallas import tpu as pltpu

_W = 5.0
_GAMMA = 0.02


def _nn_kernel(x_ref, y_ref, n_ref, out_ref, lhs_ref, *, n2, nt):
    t = pl.program_id(1)

    @pl.when(t == 0)
    def _():
        y = y_ref[0]
        nr = n_ref[0]
        sq2 = jnp.sum(y * y, axis=1, keepdims=True)
        c = jnp.sum(y * nr, axis=1, keepdims=True)
        hi = sq2.astype(jnp.bfloat16)
        lo = (sq2 - hi.astype(jnp.float32)).astype(jnp.bfloat16)
        zero = jnp.zeros((n2, 1), jnp.bfloat16)
        top = jnp.concatenate(
            [(-2.0 * y).astype(jnp.bfloat16), hi, lo], axis=1)
        bot = jnp.concatenate(
            [nr.astype(jnp.bfloat16), (-c).astype(jnp.bfloat16), zero],
            axis=1)
        lhs_ref[...] = jnp.concatenate([top, bot], axis=0)

    x = x_ref[0]
    ones = jnp.ones((x.shape[0], 2), jnp.bfloat16)
    x_aug = jnp.concatenate(
        [x.astype(jnp.bfloat16), ones], axis=1)

    both = jax.lax.dot_general(
        lhs_ref[...], x_aug, (((1,), (1,)), ((), ())),
        preferred_element_type=jnp.float32,
    )
    d = both[:n2, :]
    p = both[n2:, :]

    m = jnp.min(d, axis=0, keepdims=True)
    psel = jnp.max(jnp.where(d == m, p, -jnp.inf), axis=0)
    e = jnp.exp(_W * jnp.maximum(psel, 0.0))
    s = jnp.sum(e)

    @pl.when(t == 0)
    def _():
        out_ref[...] = jnp.zeros_like(out_ref)

    out_ref[...] += s

    @pl.when(t == nt - 1)
    def _():
        out_ref[...] *= _GAMMA


def kernel(xyz1, xyz2, nxyz2):
    b, n1, _ = xyz1.shape
    n2 = xyz2.shape[1]

    tn1 = min(512, n1)
    nt = n1 // tn1

    sums = pl.pallas_call(
        functools.partial(_nn_kernel, n2=n2, nt=nt),
        grid=(b, nt),
        in_specs=[
            pl.BlockSpec((1, tn1, 3), lambda bi, ti: (bi, ti, 0)),
            pl.BlockSpec((1, n2, 3), lambda bi, ti: (bi, 0, 0)),
            pl.BlockSpec((1, n2, 3), lambda bi, ti: (bi, 0, 0)),
        ],
        out_specs=pl.BlockSpec((1, 8, 128), lambda bi, ti: (bi, 0, 0)),
        out_shape=jax.ShapeDtypeStruct((b, 8, 128), jnp.float32),
        scratch_shapes=[pltpu.VMEM((2 * n2, 5), jnp.bfloat16)],
    )(xyz1, xyz2, nxyz2)

    return sums[:, 0, 0] / n1


# speedup vs baseline: 1.1482x; 1.1482x over previous
import functools

import jax
import jax.numpy as jnp
from jax.experimental import pallas as pl

_W = 5.0
_GAMMA = 0.02


def _nn_kernel(x_ref, rhs_ref, out_ref, *, n2, nt):
    t = pl.program_id(1)

    x = x_ref[0]
    rhs = rhs_ref[0]

    both = jax.lax.dot_general(
        x, rhs, (((1,), (0,)), ((), ())),
        preferred_element_type=jnp.float32,
    )
    d = both[:, :n2]
    p = both[:, n2:]

    m = jnp.min(d, axis=1, keepdims=True)
    psel = jnp.max(jnp.where(d == m, p, -jnp.inf), axis=1)
    e = jnp.exp(_W * jnp.maximum(psel, 0.0))
    s = jnp.sum(e)

    @pl.when(t == 0)
    def _():
        out_ref[...] = jnp.zeros_like(out_ref)

    out_ref[...] += s

    @pl.when(t == nt - 1)
    def _():
        out_ref[...] *= _GAMMA


def kernel(xyz1, xyz2, nxyz2):
    b, n1, _ = xyz1.shape
    n2 = xyz2.shape[1]

    tn1 = min(512, n1)
    nt = n1 // tn1

    x_aug = jnp.concatenate(
        [xyz1, jnp.ones((b, n1, 2), jnp.float32)],
        axis=-1).astype(jnp.bfloat16)

    y_t = jnp.transpose(xyz2, (0, 2, 1))
    n_t = jnp.transpose(nxyz2, (0, 2, 1))
    sq2 = jnp.sum(y_t * y_t, axis=1, keepdims=True)
    c = jnp.sum(y_t * n_t, axis=1, keepdims=True)
    hi = sq2.astype(jnp.bfloat16)
    lo = (sq2 - hi.astype(jnp.float32)).astype(jnp.bfloat16)
    zero = jnp.zeros_like(hi)
    rhs = jnp.concatenate([
        jnp.concatenate(
            [(-2.0 * y_t).astype(jnp.bfloat16), hi, lo], axis=1),
        jnp.concatenate(
            [n_t.astype(jnp.bfloat16), (-c).astype(jnp.bfloat16), zero],
            axis=1),
    ], axis=-1)

    sums = pl.pallas_call(
        functools.partial(_nn_kernel, n2=n2, nt=nt),
        grid=(b, nt),
        in_specs=[
            pl.BlockSpec((1, tn1, 5), lambda bi, ti: (bi, ti, 0)),
            pl.BlockSpec((1, 5, 2 * n2), lambda bi, ti: (bi, 0, 0)),
        ],
        out_specs=pl.BlockSpec((1, 8, 128), lambda bi, ti: (bi, 0, 0)),
        out_shape=jax.ShapeDtypeStruct((b, 8, 128), jnp.float32),
    )(x_aug, rhs)

    return sums[:, 0, 0] / n1
